# baseline (device time: 57310 ns/iter reference)
import jax
import jax.numpy as jnp
from jax import lax
from jax.experimental import pallas as pl
from jax.experimental.pallas import tpu as pltpu

N_DEV = 8
B, SQ, D = 2, 256, 512
H_LOC, DH = 4, 64
EPS = 1e-5

MASKS = (1, 3, 4)

PARTS = ((0, 88), (88, 88), (176, 80))
N_STAGES = len(MASKS)
N_STEPS = 2 * N_STAGES * len(PARTS)


def kernel(x, Wq, Wk, Wv, Wo, t_emb, W_mod, W_ff1, W_ff2):
    def body(x_ref, wq_ref, wk_ref, wv_ref, wo_ref, temb_ref, wmod_ref,
             wff1_ref, wff2_ref, out_ref, acc_ref, comm_ref,
             send_sems, recv_sems):
        my = lax.axis_index("i")

        barrier_sem = pltpu.get_barrier_semaphore()
        for mask in MASKS:
            pl.semaphore_signal(
                barrier_sem, inc=1,
                device_id=(my ^ mask,), device_id_type=pl.DeviceIdType.MESH,
            )
        pl.semaphore_wait(barrier_sem, len(MASKS))

        def layernorm(h):
            m = jnp.mean(h, axis=-1, keepdims=True)
            v = jnp.mean((h - m) * (h - m), axis=-1, keepdims=True)
            return (h - m) * lax.rsqrt(v + EPS)

        def start_stage(base, k, j):
            r0, rl = PARTS[j]
            step = base + k * len(PARTS) + j
            rdma = pltpu.make_async_remote_copy(
                src_ref=acc_ref.at[:, pl.ds(r0, rl), :],
                dst_ref=comm_ref.at[step, :, pl.ds(0, rl), :],
                send_sem=send_sems.at[step],
                recv_sem=recv_sems.at[step],
                device_id=(my ^ MASKS[(k + 2 * j) % 3],),
                device_id_type=pl.DeviceIdType.MESH,
            )
            rdma.start()
            return rdma

        def add_part(base, k, j):
            r0, rl = PARTS[j]
            step = base + k * len(PARTS) + j
            acc_ref[:, r0:r0 + rl, :] = (
                acc_ref[:, r0:r0 + rl, :] + comm_ref[step, :, 0:rl, :]
            )

        def finish_ar(base, pending, on_part_done=None):
            for k in range(1, N_STAGES):
                for j in range(len(PARTS)):
                    pending[j].wait()
                    add_part(base, k - 1, j)
                    pending[j] = start_stage(base, k, j)
            for j in range(len(PARTS)):
                pending[j].wait()
                add_part(base, N_STAGES - 1, j)
                if on_part_done is not None:
                    on_part_done(j)

        mod = jnp.dot(temb_ref[...], wmod_ref[...],
                      preferred_element_type=jnp.float32)
        sa, sha, ga, sm, shm, gm = [mod[:, i * D:(i + 1) * D] for i in range(6)]

        ar1_base = 0
        ar2_base = N_STAGES * len(PARTS)

        qkv = []
        for b in range(B):
            xb = x_ref[b]
            xa = layernorm(xb) * (1.0 + sa[b:b + 1, :]) + sha[b:b + 1, :]
            q = jnp.dot(xa, wq_ref[...], preferred_element_type=jnp.float32)
            k = jnp.dot(xa, wk_ref[...], preferred_element_type=jnp.float32)
            v = jnp.dot(xa, wv_ref[...], preferred_element_type=jnp.float32)
            qkv.append((q, k, v))

        pending = [None] * len(PARTS)
        for j, (r0, rl) in enumerate(PARTS):
            for b in range(B):
                q, k, v = qkv[b]
                outs = []
                for h in range(H_LOC):
                    sl = slice(h * DH, (h + 1) * DH)
                    s = lax.dot_general(
                        q[r0:r0 + rl, sl], k[:, sl], (((1,), (1,)), ((), ())),
                        preferred_element_type=jnp.float32) * 0.125
                    s = s - jnp.max(s, axis=-1, keepdims=True)
                    p = jnp.exp(s)
                    p = p / jnp.sum(p, axis=-1, keepdims=True)
                    outs.append(jnp.dot(p, v[:, sl],
                                        preferred_element_type=jnp.float32))
                o = jnp.concatenate(outs, axis=1)
                acc_ref[b, r0:r0 + rl, :] = jnp.dot(
                    o, wo_ref[...], preferred_element_type=jnp.float32)
            pending[j] = start_stage(ar1_base, 0, j)

        ff_pending = [None] * len(PARTS)

        def ffn_part(j):
            r0, rl = PARTS[j]
            for b in range(B):
                x1 = x_ref[b, r0:r0 + rl, :] + ga[b:b + 1, :] * acc_ref[b, r0:r0 + rl, :]
                out_ref[b, r0:r0 + rl, :] = x1
                xm = layernorm(x1) * (1.0 + sm[b:b + 1, :]) + shm[b:b + 1, :]
                hh = jnp.dot(xm, wff1_ref[...],
                             preferred_element_type=jnp.float32)
                hh = hh / (1.0 + jnp.exp(-hh))
                acc_ref[b, r0:r0 + rl, :] = jnp.dot(
                    hh, wff2_ref[...], preferred_element_type=jnp.float32)
            ff_pending[j] = start_stage(ar2_base, 0, j)

        finish_ar(ar1_base, pending, on_part_done=ffn_part)

        def final_part(j):
            r0, rl = PARTS[j]
            for b in range(B):
                out_ref[b, r0:r0 + rl, :] = (
                    out_ref[b, r0:r0 + rl, :]
                    + gm[b:b + 1, :] * acc_ref[b, r0:r0 + rl, :]
                )

        finish_ar(ar2_base, ff_pending, on_part_done=final_part)

    max_rows = max(rl for _, rl in PARTS)
    return pl.pallas_call(
        body,
        out_shape=jax.ShapeDtypeStruct((B, SQ, D), jnp.float32),
        in_specs=[pl.BlockSpec(memory_space=pltpu.VMEM)] * 9,
        out_specs=pl.BlockSpec(memory_space=pltpu.VMEM),
        scratch_shapes=[
            pltpu.VMEM((B, SQ, D), jnp.float32),
            pltpu.VMEM((N_STEPS, B, max_rows, D), jnp.float32),
            pltpu.SemaphoreType.DMA((N_STEPS,)),
            pltpu.SemaphoreType.DMA((N_STEPS,)),
        ],
        compiler_params=pltpu.CompilerParams(collective_id=0),
    )(x, Wq, Wk, Wv, Wo, t_emb, W_mod, W_ff1, W_ff2)


# device time: 53759 ns/iter; 1.0661x vs baseline; 1.0661x over previous
import jax
import jax.numpy as jnp
from jax import lax
from jax.experimental import pallas as pl
from jax.experimental.pallas import tpu as pltpu

N_DEV = 8
B, SQ, D = 2, 256, 512
H_LOC, DH = 4, 64
EPS = 1e-5

MASKS = (1, 3, 4)

PARTS = ((0, 88), (88, 88), (176, 80))
N_STAGES = len(MASKS)
N_STEPS = 2 * N_STAGES * len(PARTS)


def kernel(x, Wq, Wk, Wv, Wo, t_emb, W_mod, W_ff1, W_ff2):
    def body(x_ref, wq_ref, wk_ref, wv_ref, wo_ref, temb_ref, wmod_ref,
             wff1_ref, wff2_ref, out_ref, acc_ref, comm_ref,
             send_sems, recv_sems):
        my = lax.axis_index("i")

        barrier_sem = pltpu.get_barrier_semaphore()
        for mask in MASKS:
            pl.semaphore_signal(
                barrier_sem, inc=1,
                device_id=(my ^ mask,), device_id_type=pl.DeviceIdType.MESH,
            )
        pl.semaphore_wait(barrier_sem, len(MASKS))

        def layernorm(h):
            m = jnp.mean(h, axis=-1, keepdims=True)
            v = jnp.mean((h - m) * (h - m), axis=-1, keepdims=True)
            return (h - m) * lax.rsqrt(v + EPS)

        def start_stage(base, k, j):
            r0, rl = PARTS[j]
            step = base + k * len(PARTS) + j
            rdma = pltpu.make_async_remote_copy(
                src_ref=acc_ref.at[:, pl.ds(r0, rl), :],
                dst_ref=comm_ref.at[step, :, pl.ds(0, rl), :],
                send_sem=send_sems.at[step],
                recv_sem=recv_sems.at[step],
                device_id=(my ^ MASKS[(k + 2 * j) % 3],),
                device_id_type=pl.DeviceIdType.MESH,
            )
            rdma.start()
            return rdma

        def add_part(base, k, j):
            r0, rl = PARTS[j]
            step = base + k * len(PARTS) + j
            acc_ref[:, r0:r0 + rl, :] = (
                acc_ref[:, r0:r0 + rl, :] + comm_ref[step, :, 0:rl, :]
            )

        def finish_ar(base, pending, on_part_done=None):
            for k in range(1, N_STAGES):
                for j in range(len(PARTS)):
                    pending[j].wait()
                    add_part(base, k - 1, j)
                    pending[j] = start_stage(base, k, j)
            for j in range(len(PARTS)):
                pending[j].wait()
                add_part(base, N_STAGES - 1, j)
                if on_part_done is not None:
                    on_part_done(j)

        mod = jnp.dot(temb_ref[...], wmod_ref[...],
                      preferred_element_type=jnp.float32)
        sa, sha, ga, sm, shm, gm = [mod[:, i * D:(i + 1) * D] for i in range(6)]

        ar1_base = 0
        ar2_base = N_STAGES * len(PARTS)

        for b in range(B):
            xb = x_ref[b]
            xa = layernorm(xb) * (1.0 + sa[b:b + 1, :]) + sha[b:b + 1, :]
            q = jnp.dot(xa, wq_ref[...], preferred_element_type=jnp.float32)
            k = jnp.dot(xa, wk_ref[...], preferred_element_type=jnp.float32)
            v = jnp.dot(xa, wv_ref[...], preferred_element_type=jnp.float32)
            outs = []
            for h in range(H_LOC):
                sl = slice(h * DH, (h + 1) * DH)
                s = lax.dot_general(
                    q[:, sl], k[:, sl], (((1,), (1,)), ((), ())),
                    preferred_element_type=jnp.float32) * 0.125
                s = s - jnp.max(s, axis=-1, keepdims=True)
                p = jnp.exp(s)
                p = p / jnp.sum(p, axis=-1, keepdims=True)
                outs.append(jnp.dot(p, v[:, sl],
                                    preferred_element_type=jnp.float32))
            o = jnp.concatenate(outs, axis=1)
            acc_ref[b] = jnp.dot(o, wo_ref[...],
                                 preferred_element_type=jnp.float32)

        pending = [start_stage(ar1_base, 0, j) for j in range(len(PARTS))]

        ff_pending = [None] * len(PARTS)

        def ffn_part(j):
            r0, rl = PARTS[j]
            for b in range(B):
                x1 = x_ref[b, r0:r0 + rl, :] + ga[b:b + 1, :] * acc_ref[b, r0:r0 + rl, :]
                out_ref[b, r0:r0 + rl, :] = x1
                xm = layernorm(x1) * (1.0 + sm[b:b + 1, :]) + shm[b:b + 1, :]
                hh = jnp.dot(xm, wff1_ref[...],
                             preferred_element_type=jnp.float32)
                hh = hh / (1.0 + jnp.exp(-hh))
                acc_ref[b, r0:r0 + rl, :] = jnp.dot(
                    hh, wff2_ref[...], preferred_element_type=jnp.float32)
            ff_pending[j] = start_stage(ar2_base, 0, j)

        finish_ar(ar1_base, pending, on_part_done=ffn_part)

        def final_part(j):
            r0, rl = PARTS[j]
            for b in range(B):
                out_ref[b, r0:r0 + rl, :] = (
                    out_ref[b, r0:r0 + rl, :]
                    + gm[b:b + 1, :] * acc_ref[b, r0:r0 + rl, :]
                )

        finish_ar(ar2_base, ff_pending, on_part_done=final_part)

    max_rows = max(rl for _, rl in PARTS)
    return pl.pallas_call(
        body,
        out_shape=jax.ShapeDtypeStruct((B, SQ, D), jnp.float32),
        in_specs=[pl.BlockSpec(memory_space=pltpu.VMEM)] * 9,
        out_specs=pl.BlockSpec(memory_space=pltpu.VMEM),
        scratch_shapes=[
            pltpu.VMEM((B, SQ, D), jnp.float32),
            pltpu.VMEM((N_STEPS, B, max_rows, D), jnp.float32),
            pltpu.SemaphoreType.DMA((N_STEPS,)),
            pltpu.SemaphoreType.DMA((N_STEPS,)),
        ],
        compiler_params=pltpu.CompilerParams(collective_id=0),
    )(x, Wq, Wk, Wv, Wo, t_emb, W_mod, W_ff1, W_ff2)
